# baseline (speedup 1.0000x reference)
import jax
import jax.numpy as jnp
from jax.experimental import pallas as pl
from jax.experimental.pallas import tpu as pltpu

_EPS = 1e-12


def _vlad_body(x_ref, w_ref, c_ref, o_ref):
    xb = x_ref[0]
    C = xb.shape[0]

    sq = xb * xb
    nrm2 = jnp.dot(jnp.ones((1, C), jnp.bfloat16), sq,
                   preferred_element_type=jnp.float32)
    rinv = 1.0 / jnp.maximum(jnp.sqrt(nrm2), _EPS)

    raw = jnp.dot(w_ref[...], xb,
                  preferred_element_type=jnp.float32)
    logits = raw * rinv

    m = jnp.max(logits, axis=0, keepdims=True)
    e = jnp.exp(logits - m)
    a = e / jnp.sum(e, axis=0, keepdims=True)

    asum = jnp.sum(a, axis=1, keepdims=True)
    ab = (a * rinv).astype(jnp.bfloat16)
    vlad = jax.lax.dot_general(
        ab, xb, (((1,), (1,)), ((), ())),
        preferred_element_type=jnp.float32)
    vlad = vlad - asum * c_ref[...]

    rn2 = jnp.sum(vlad * vlad, axis=1, keepdims=True)
    vlad = vlad / jnp.maximum(jnp.sqrt(rn2), _EPS)

    gn2 = jnp.sum(vlad * vlad, keepdims=True)
    o_ref[0] = vlad / jnp.maximum(jnp.sqrt(gn2), _EPS)


def kernel(x, conv_w, centroids):
    N, C, H, W = x.shape
    K = conv_w.shape[0]
    S = H * W
    xh = x.astype(jnp.bfloat16).reshape(N, C, S)
    wh = conv_w.astype(jnp.bfloat16)

    out = pl.pallas_call(
        _vlad_body,
        grid=(N,),
        in_specs=[
            pl.BlockSpec((1, C, S), lambda n: (n, 0, 0)),
            pl.BlockSpec((K, C), lambda n: (0, 0)),
            pl.BlockSpec((K, C), lambda n: (0, 0)),
        ],
        out_specs=pl.BlockSpec((1, K, C), lambda n: (n, 0, 0)),
        out_shape=jax.ShapeDtypeStruct((N, K, C), jnp.float32),
        compiler_params=pltpu.CompilerParams(
            dimension_semantics=("parallel",),
        ),
    )(xh, wh, centroids)
    return out.reshape(N, K * C)


# speedup vs baseline: 1.1878x; 1.1878x over previous
import jax
import jax.numpy as jnp
from jax.experimental import pallas as pl
from jax.experimental.pallas import tpu as pltpu

_EPS = 1e-12
_BATCH = 4


def _vlad_one(xb_f32, w_ref, c_ref):
    C = xb_f32.shape[0]
    xb = xb_f32.astype(jnp.bfloat16)

    sq = xb * xb
    nrm2 = jnp.dot(jnp.ones((1, C), jnp.bfloat16), sq,
                   preferred_element_type=jnp.float32)
    nrm = jnp.maximum(jnp.sqrt(nrm2), _EPS)
    rinv = 1.0 / nrm

    raw = jnp.dot(w_ref[...], xb,
                  preferred_element_type=jnp.float32)
    e = jnp.exp(raw * rinv)
    scale = rinv / jnp.sum(e, axis=0, keepdims=True)
    ab = (e * scale).astype(jnp.bfloat16)

    vlad = jax.lax.dot_general(
        ab, xb, (((1,), (1,)), ((), ())),
        preferred_element_type=jnp.float32)
    nrm8 = jnp.broadcast_to(nrm, (8, nrm.shape[1])).astype(jnp.bfloat16)
    asum = jax.lax.dot_general(
        ab, nrm8, (((1,), (1,)), ((), ())),
        preferred_element_type=jnp.float32)[:, 0:1]
    vlad = vlad - asum * c_ref[...]

    rn2 = jnp.sum(vlad * vlad, axis=1, keepdims=True)
    vlad = vlad / jnp.maximum(jnp.sqrt(rn2), _EPS)

    gn2 = jnp.sum(vlad * vlad, keepdims=True)
    return vlad / jnp.maximum(jnp.sqrt(gn2), _EPS)


def _vlad_body(x_ref, w_ref, c_ref, o_ref):
    for i in range(_BATCH):
        o_ref[i] = _vlad_one(x_ref[i], w_ref, c_ref)


def kernel(x, conv_w, centroids):
    N, C, H, W = x.shape
    K = conv_w.shape[0]
    S = H * W
    xf = x.reshape(N, C, S)
    wh = conv_w.astype(jnp.bfloat16)

    out = pl.pallas_call(
        _vlad_body,
        grid=(N // _BATCH,),
        in_specs=[
            pl.BlockSpec((_BATCH, C, S), lambda n: (n, 0, 0)),
            pl.BlockSpec((K, C), lambda n: (0, 0)),
            pl.BlockSpec((K, C), lambda n: (0, 0)),
        ],
        out_specs=pl.BlockSpec((_BATCH, K, C), lambda n: (n, 0, 0)),
        out_shape=jax.ShapeDtypeStruct((N, K, C), jnp.float32),
        compiler_params=pltpu.CompilerParams(
            dimension_semantics=("parallel",),
        ),
    )(xf, wh, centroids)
    return out.reshape(N, K * C)


# speedup vs baseline: 1.2114x; 1.0199x over previous
import jax
import jax.numpy as jnp
from jax.experimental import pallas as pl
from jax.experimental.pallas import tpu as pltpu

_EPS = 1e-12
_BATCH = 8


def _vlad_one(xb_f32, wb, c_ref):
    C = xb_f32.shape[0]
    xb = xb_f32.astype(jnp.bfloat16)

    sq = xb * xb
    nrm2 = jnp.dot(jnp.ones((1, C), jnp.bfloat16), sq,
                   preferred_element_type=jnp.float32)
    nrm = jnp.maximum(jnp.sqrt(nrm2), _EPS)
    rinv = 1.0 / nrm

    raw = jnp.dot(wb, xb,
                  preferred_element_type=jnp.float32)
    e = jnp.exp(raw * rinv)
    scale = rinv / jnp.sum(e, axis=0, keepdims=True)
    ab = (e * scale).astype(jnp.bfloat16)

    vlad = jax.lax.dot_general(
        ab, xb, (((1,), (1,)), ((), ())),
        preferred_element_type=jnp.float32)
    nrm8 = jnp.broadcast_to(nrm, (8, nrm.shape[1])).astype(jnp.bfloat16)
    asum = jax.lax.dot_general(
        ab, nrm8, (((1,), (1,)), ((), ())),
        preferred_element_type=jnp.float32)[:, 0:1]
    vlad = vlad - asum * c_ref[...]

    rn2 = jnp.sum(vlad * vlad, axis=1, keepdims=True)
    vlad = vlad / jnp.maximum(jnp.sqrt(rn2), _EPS)

    gn2 = jnp.sum(vlad * vlad, keepdims=True)
    return vlad / jnp.maximum(jnp.sqrt(gn2), _EPS)


def _vlad_body(x_ref, w_ref, c_ref, o_ref):
    wb = w_ref[...].astype(jnp.bfloat16)
    for i in range(_BATCH):
        o_ref[i] = _vlad_one(x_ref[i], wb, c_ref)


def kernel(x, conv_w, centroids):
    N, C, H, W = x.shape
    K = conv_w.shape[0]
    S = H * W
    xf = x.reshape(N, C, S)

    out = pl.pallas_call(
        _vlad_body,
        grid=(N // _BATCH,),
        in_specs=[
            pl.BlockSpec((_BATCH, C, S), lambda n: (n, 0, 0)),
            pl.BlockSpec((K, C), lambda n: (0, 0)),
            pl.BlockSpec((K, C), lambda n: (0, 0)),
        ],
        out_specs=pl.BlockSpec((_BATCH, K, C), lambda n: (n, 0, 0)),
        out_shape=jax.ShapeDtypeStruct((N, K, C), jnp.float32),
        compiler_params=pltpu.CompilerParams(
            dimension_semantics=("parallel",),
            vmem_limit_bytes=50 * 1024 * 1024,
        ),
    )(xf, conv_w, centroids)
    return out.reshape(N, K * C)
